# Initial kernel scaffold; baseline (speedup 1.0000x reference)
import jax
import jax.numpy as jnp
from jax.experimental import pallas as pl


def kernel(x, edge_index, edge_type, emb_w, basis1, comp1, root1, bias1, basis2, comp2, root2, bias2, cls_w):
    raise NotImplementedError("write your pallas kernel here")


# speedup vs baseline: 3.8637x; 3.8637x over previous
import functools

import jax
import jax.numpy as jnp
from jax import lax
from jax.experimental import pallas as pl
from jax.experimental.pallas import tpu as pltpu
from jax.experimental.pallas import tpu_sc as plsc

N = 10000
E = 320000
R = 8
D = 128
C = 40

NC, NS, LANES = 2, 16, 16
NW = NC * NS
EPW = E // NW
CHW = 80
EROWS = E // CHW
ROWS_W = EPW // CHW
PB = 25
NR = N * R

MB = 1000
GM = N // MB
KB = 1000


def _wmix_body(comp_ref, basisf_ref, rootf_ref, o_ref):
    o_ref[0:R, :] = jnp.dot(comp_ref[...], basisf_ref[...],
                            preferred_element_type=jnp.float32)
    o_ref[R:R + 1, :] = rootf_ref[...]


def _wmix(comp, basisf, rootf):
    return pl.pallas_call(
        _wmix_body,
        out_shape=jax.ShapeDtypeStruct((R + 1, D * D), jnp.float32),
    )(comp, basisf, rootf)


def _embed_body(x_ref, w_ref, o_ref):
    o_ref[...] = jnp.maximum(
        jnp.dot(x_ref[...], w_ref[...], preferred_element_type=jnp.float32),
        0.0)


EMB_MB = 400


def _embed_mm(x, emb_w):
    return pl.pallas_call(
        _embed_body,
        grid=(N // EMB_MB,),
        in_specs=[pl.BlockSpec((EMB_MB, N), lambda m: (m, 0)),
                  pl.BlockSpec((N, D), lambda m: (0, 0))],
        out_specs=pl.BlockSpec((EMB_MB, D), lambda m: (m, 0)),
        out_shape=jax.ShapeDtypeStruct((N, D), jnp.float32),
    )(x, emb_w)


def _layer_mm_body(h_ref, wf_ref, bias_ref, xw_ref, hr_ref):
    h = h_ref[...]
    for r in range(R):
        xw_ref[r] = jnp.dot(h, wf_ref[r], preferred_element_type=jnp.float32)
    hr_ref[...] = (jnp.dot(h, wf_ref[R], preferred_element_type=jnp.float32)
                   + bias_ref[...])


def _layer_mm(h, wf3, bias2d):
    return pl.pallas_call(
        _layer_mm_body,
        grid=(GM,),
        in_specs=[pl.BlockSpec((MB, D), lambda m: (m, 0)),
                  pl.BlockSpec((R + 1, D, D), lambda m: (0, 0, 0)),
                  pl.BlockSpec((1, D), lambda m: (0, 0))],
        out_specs=[pl.BlockSpec((R, MB, D), lambda m: (0, m, 0)),
                   pl.BlockSpec((MB, D), lambda m: (m, 0))],
        out_shape=[jax.ShapeDtypeStruct((R, N, D), jnp.float32),
                   jax.ShapeDtypeStruct((N, D), jnp.float32)],
    )(h, wf3, bias2d)


def _layer_mm2_body(agg0_ref, agg1_ref, hrp_ref, wf_ref, bias_ref,
                    xw_ref, hr_ref):
    h = jnp.maximum(agg0_ref[...] + agg1_ref[...] + hrp_ref[...], 0.0)
    for r in range(R):
        xw_ref[r] = jnp.dot(h, wf_ref[r], preferred_element_type=jnp.float32)
    hr_ref[...] = (jnp.dot(h, wf_ref[R], preferred_element_type=jnp.float32)
                   + bias_ref[...])


def _layer_mm2(agg_flat, hr_prev, wf3, bias2d):
    return pl.pallas_call(
        _layer_mm2_body,
        grid=(GM,),
        in_specs=[pl.BlockSpec((MB, D), lambda m: (m, 0)),
                  pl.BlockSpec((MB, D), lambda m: (m + GM, 0)),
                  pl.BlockSpec((MB, D), lambda m: (m, 0)),
                  pl.BlockSpec((R + 1, D, D), lambda m: (0, 0, 0)),
                  pl.BlockSpec((1, D), lambda m: (0, 0))],
        out_specs=[pl.BlockSpec((R, MB, D), lambda m: (0, m, 0)),
                   pl.BlockSpec((MB, D), lambda m: (m, 0))],
        out_shape=[jax.ShapeDtypeStruct((R, N, D), jnp.float32),
                   jax.ShapeDtypeStruct((N, D), jnp.float32)],
    )(agg_flat, agg_flat, hr_prev, wf3, bias2d)


def _decode_body(agg0_ref, agg1_ref, hrp_ref, cls_ref, o_ref):
    h = jnp.maximum(agg0_ref[...] + agg1_ref[...] + hrp_ref[...], 0.0)
    o_ref[...] = jnp.dot(h, cls_ref[...], preferred_element_type=jnp.float32)


def _decode(agg_flat, hr_prev, cls_w):
    return pl.pallas_call(
        _decode_body,
        grid=(GM,),
        in_specs=[pl.BlockSpec((MB, D), lambda m: (m, 0)),
                  pl.BlockSpec((MB, D), lambda m: (m + GM, 0)),
                  pl.BlockSpec((MB, D), lambda m: (m, 0)),
                  pl.BlockSpec((D, C), lambda m: (0, 0))],
        out_specs=pl.BlockSpec((MB, C), lambda m: (m, 0)),
        out_shape=jax.ShapeDtypeStruct((N, C), jnp.float32),
    )(agg_flat, agg_flat, hr_prev, cls_w)


@functools.cache
def _mesh():
    return plsc.VectorSubcoreMesh(core_axis_name="c", subcore_axis_name="s",
                                  num_cores=NC, num_subcores=NS)


def _norm_edges(src2, dst2, et2):
    @functools.partial(
        pl.kernel,
        out_type=[jax.ShapeDtypeStruct((EROWS, CHW), jnp.float32),
                  jax.ShapeDtypeStruct((EROWS, CHW), jnp.int32)],
        mesh=_mesh(),
        compiler_params=pltpu.CompilerParams(use_tc_tiling_on_sc=False, needs_layout_passes=False),
        scratch_types=[
            pltpu.VMEM_SHARED((NR,), jnp.float32),
            pltpu.VMEM((NR,), jnp.float32),
            pltpu.VMEM((PB, CHW), jnp.int32),
            pltpu.VMEM((PB, CHW), jnp.int32),
            pltpu.VMEM((PB, CHW), jnp.int32),
            pltpu.VMEM((PB, CHW), jnp.int32),
            pltpu.VMEM((PB, CHW), jnp.float32),
            pltpu.VMEM((PB, CHW), jnp.int32),
            pltpu.VMEM((CHW,), jnp.float32),
            pltpu.VMEM((PB * CHW,), jnp.float32),
        ],
    )
    def k(src_h, dst_h, et_h, norm_h, gidx_h,
          cnt_sh, cnt_v, dst_v, et_v, src_v, keyv_v, norm_v, gidx_v, ones_v,
          zf_v):
        sid = lax.axis_index("s")
        cid = lax.axis_index("c")

        for j in range(CHW // LANES):
            ones_v[pl.ds(j * LANES, LANES)] = jnp.ones((LANES,), jnp.float32)

        zwords = PB * CHW

        def zrow(i, _):
            zf_v[pl.ds(i * LANES, LANES)] = jnp.zeros((LANES,), jnp.float32)
            return 0
        lax.fori_loop(0, zwords // LANES, zrow, 0)
        for t in range(3):
            chunk_id = sid + NS * t

            @pl.when(chunk_id < NR // zwords)
            def _():
                pltpu.sync_copy(zf_v,
                                cnt_sh.at[pl.ds(chunk_id * zwords, zwords)])
        plsc.subcore_barrier()

        rows_all = EROWS // NS

        def count_pass(p, _):
            base = sid * rows_all + p * PB
            pltpu.sync_copy(dst_h.at[pl.ds(base, PB)], dst_v)
            pltpu.sync_copy(et_h.at[pl.ds(base, PB)], et_v)

            def kvrow(i, _2):
                for j in range(CHW // LANES):
                    dd = dst_v[i, pl.ds(j * LANES, LANES)]
                    tt = et_v[i, pl.ds(j * LANES, LANES)]
                    keyv_v[i, pl.ds(j * LANES, LANES)] = dd * R + tt
                return 0
            lax.fori_loop(0, PB, kvrow, 0)

            def scat(i, _2):
                pltpu.sync_copy(ones_v, cnt_sh.at[keyv_v.at[i]], add=True)
                return 0
            lax.fori_loop(0, PB, scat, 0)
            return 0
        lax.fori_loop(0, rows_all // PB, count_pass, 0)
        plsc.subcore_barrier()

        pltpu.sync_copy(cnt_sh, cnt_v)
        wbase = (cid * NS + sid) * ROWS_W

        for p in range(ROWS_W // PB):
            base = wbase + p * PB
            pltpu.sync_copy(dst_h.at[pl.ds(base, PB)], dst_v)
            pltpu.sync_copy(et_h.at[pl.ds(base, PB)], et_v)
            pltpu.sync_copy(src_h.at[pl.ds(base, PB)], src_v)

            def nrow(i, _2):
                for j in range(CHW // LANES):
                    dd = dst_v[i, pl.ds(j * LANES, LANES)]
                    tt = et_v[i, pl.ds(j * LANES, LANES)]
                    ss = src_v[i, pl.ds(j * LANES, LANES)]
                    cnt = plsc.load_gather(cnt_v, [dd * R + tt])
                    norm_v[i, pl.ds(j * LANES, LANES)] = (
                        1.0 / jnp.maximum(cnt, 1.0))
                    gidx_v[i, pl.ds(j * LANES, LANES)] = tt * N + ss
                return 0
            lax.fori_loop(0, PB, nrow, 0)
            pltpu.sync_copy(norm_v, norm_h.at[pl.ds(base, PB)])
            pltpu.sync_copy(gidx_v, gidx_h.at[pl.ds(base, PB)])

    return k(src2, dst2, et2)


_ZROWS = 25


def _edge_scatter(xwf, gidx2, norm2, dst2):
    @functools.partial(
        pl.kernel,
        out_type=jax.ShapeDtypeStruct((NC * N, D), jnp.float32),
        mesh=_mesh(),
        compiler_params=pltpu.CompilerParams(use_tc_tiling_on_sc=False, needs_layout_passes=False),
        scratch_types=[
            pltpu.VMEM_SHARED((N, D), jnp.float32),
            pltpu.VMEM((ROWS_W, CHW), jnp.int32),
            pltpu.VMEM((ROWS_W, CHW), jnp.float32),
            pltpu.VMEM((ROWS_W, CHW), jnp.int32),
            pltpu.VMEM((CHW, D), jnp.float32),
            pltpu.VMEM((_ZROWS, D), jnp.float32),
            pltpu.SemaphoreType.DMA,
        ],
    )
    def k(xw_h, gi_h, no_h, ds_h, out_h,
          agg_sh, gi_v, no_v, ds_v, rows_v, z_v, sem):
        sid = lax.axis_index("s")
        cid = lax.axis_index("c")
        nrows = N // NS

        def zrow(i, _):
            for j in range(D // LANES):
                z_v[i, pl.ds(j * LANES, LANES)] = jnp.zeros((LANES,),
                                                            jnp.float32)
            return 0
        lax.fori_loop(0, _ZROWS, zrow, 0)
        for t in range(nrows // _ZROWS):
            pltpu.sync_copy(
                z_v, agg_sh.at[pl.ds(sid * nrows + t * _ZROWS, _ZROWS)])
        plsc.subcore_barrier()

        w = cid * NS + sid
        pltpu.sync_copy(gi_h.at[pl.ds(w * ROWS_W, ROWS_W)], gi_v)
        pltpu.sync_copy(no_h.at[pl.ds(w * ROWS_W, ROWS_W)], no_v)
        pltpu.sync_copy(ds_h.at[pl.ds(w * ROWS_W, ROWS_W)], ds_v)

        iota16 = lax.iota(jnp.int32, LANES)

        def chunk(i, _):
            pltpu.async_copy(xw_h.at[gi_v.at[i]], rows_v, sem).wait()
            nvs = tuple(no_v[i, pl.ds(j * LANES, LANES)]
                        for j in range(CHW // LANES))

            def scale_col(cc, carry):
                colv = jnp.full((LANES,), cc, jnp.int32)
                for j in range(CHW // LANES):
                    ridx = iota16 + (j * LANES)
                    vals = plsc.load_gather(rows_v, [ridx, colv])
                    plsc.store_scatter(rows_v, [ridx, colv], vals * carry[j])
                return carry
            lax.fori_loop(0, D, scale_col, nvs)
            pltpu.sync_copy(rows_v, agg_sh.at[ds_v.at[i]], add=True)
            return 0
        lax.fori_loop(0, ROWS_W, chunk, 0)
        plsc.subcore_barrier()
        pltpu.sync_copy(agg_sh.at[pl.ds(sid * nrows, nrows)],
                        out_h.at[pl.ds(cid * N + sid * nrows, nrows)])

    return k(xwf, gidx2, norm2, dst2)


def kernel(x, edge_index, edge_type, emb_w, basis1, comp1, root1, bias1,
           basis2, comp2, root2, bias2, cls_w):
    src2 = edge_index[0].reshape(EROWS, CHW)
    dst2 = edge_index[1].reshape(EROWS, CHW)
    et2 = edge_type.reshape(EROWS, CHW)

    wf1 = _wmix(comp1, basis1.reshape(R, D * D),
                root1.reshape(1, D * D)).reshape(R + 1, D, D)
    wf2 = _wmix(comp2, basis2.reshape(R, D * D),
                root2.reshape(1, D * D)).reshape(R + 1, D, D)

    norm2, gidx2 = _norm_edges(src2, dst2, et2)

    h0 = _embed_mm(x, emb_w)
    xw1, hr1 = _layer_mm(h0, wf1, bias1.reshape(1, D))
    agg1 = _edge_scatter(xw1.reshape(R * N, D), gidx2, norm2, dst2)
    xw2, hr2 = _layer_mm2(agg1, hr1, wf2, bias2.reshape(1, D))
    agg2 = _edge_scatter(xw2.reshape(R * N, D), gidx2, norm2, dst2)
    return _decode(agg2, hr2, cls_w)


# speedup vs baseline: 17.1303x; 4.4336x over previous
import functools

import jax
import jax.numpy as jnp
from jax import lax
from jax.experimental import pallas as pl
from jax.experimental.pallas import tpu as pltpu
from jax.experimental.pallas import tpu_sc as plsc

N = 10000
E = 320000
R = 8
D = 128
C = 40

NC, NS, LANES = 2, 16, 16
NW = NC * NS
EPW = E // NW
CHW = 80
EROWS = E // CHW
ROWS_W = EPW // CHW
PB = 25
NR = N * R

MB = 1000
GM = N // MB
KB = 1000


def _wmix_body(comp_ref, basisf_ref, rootf_ref, o_ref):
    o_ref[0:R, :] = jnp.dot(comp_ref[...], basisf_ref[...],
                            preferred_element_type=jnp.float32)
    o_ref[R:R + 1, :] = rootf_ref[...]


def _wmix(comp, basisf, rootf):
    return pl.pallas_call(
        _wmix_body,
        out_shape=jax.ShapeDtypeStruct((R + 1, D * D), jnp.float32),
    )(comp, basisf, rootf)


def _embed_body(x_ref, w_ref, o_ref):
    o_ref[...] = jnp.maximum(
        jnp.dot(x_ref[...], w_ref[...], preferred_element_type=jnp.float32),
        0.0)


EMB_MB = 400


def _embed_mm(x, emb_w):
    return pl.pallas_call(
        _embed_body,
        grid=(N // EMB_MB,),
        in_specs=[pl.BlockSpec((EMB_MB, N), lambda m: (m, 0)),
                  pl.BlockSpec((N, D), lambda m: (0, 0))],
        out_specs=pl.BlockSpec((EMB_MB, D), lambda m: (m, 0)),
        out_shape=jax.ShapeDtypeStruct((N, D), jnp.float32),
    )(x, emb_w)


def _layer_mm_body(h_ref, wf_ref, bias_ref, xw_ref, hr_ref):
    h = h_ref[...]
    for r in range(R):
        xw_ref[r] = jnp.dot(h, wf_ref[r], preferred_element_type=jnp.float32)
    hr_ref[...] = (jnp.dot(h, wf_ref[R], preferred_element_type=jnp.float32)
                   + bias_ref[...])


def _layer_mm(h, wf3, bias2d):
    return pl.pallas_call(
        _layer_mm_body,
        grid=(GM,),
        in_specs=[pl.BlockSpec((MB, D), lambda m: (m, 0)),
                  pl.BlockSpec((R + 1, D, D), lambda m: (0, 0, 0)),
                  pl.BlockSpec((1, D), lambda m: (0, 0))],
        out_specs=[pl.BlockSpec((R, MB, D), lambda m: (0, m, 0)),
                   pl.BlockSpec((MB, D), lambda m: (m, 0))],
        out_shape=[jax.ShapeDtypeStruct((R, N, D), jnp.float32),
                   jax.ShapeDtypeStruct((N, D), jnp.float32)],
    )(h, wf3, bias2d)


def _layer_mm2_body(agg0_ref, agg1_ref, hrp_ref, wf_ref, bias_ref,
                    xw_ref, hr_ref):
    h = jnp.maximum(agg0_ref[...] + agg1_ref[...] + hrp_ref[...], 0.0)
    for r in range(R):
        xw_ref[r] = jnp.dot(h, wf_ref[r], preferred_element_type=jnp.float32)
    hr_ref[...] = (jnp.dot(h, wf_ref[R], preferred_element_type=jnp.float32)
                   + bias_ref[...])


def _layer_mm2(agg_flat, hr_prev, wf3, bias2d):
    return pl.pallas_call(
        _layer_mm2_body,
        grid=(GM,),
        in_specs=[pl.BlockSpec((MB, D), lambda m: (m, 0)),
                  pl.BlockSpec((MB, D), lambda m: (m + GM, 0)),
                  pl.BlockSpec((MB, D), lambda m: (m, 0)),
                  pl.BlockSpec((R + 1, D, D), lambda m: (0, 0, 0)),
                  pl.BlockSpec((1, D), lambda m: (0, 0))],
        out_specs=[pl.BlockSpec((R, MB, D), lambda m: (0, m, 0)),
                   pl.BlockSpec((MB, D), lambda m: (m, 0))],
        out_shape=[jax.ShapeDtypeStruct((R, N, D), jnp.float32),
                   jax.ShapeDtypeStruct((N, D), jnp.float32)],
    )(agg_flat, agg_flat, hr_prev, wf3, bias2d)


def _decode_body(agg0_ref, agg1_ref, hrp_ref, cls_ref, o_ref):
    h = jnp.maximum(agg0_ref[...] + agg1_ref[...] + hrp_ref[...], 0.0)
    o_ref[...] = jnp.dot(h, cls_ref[...], preferred_element_type=jnp.float32)


def _decode(agg_flat, hr_prev, cls_w):
    return pl.pallas_call(
        _decode_body,
        grid=(GM,),
        in_specs=[pl.BlockSpec((MB, D), lambda m: (m, 0)),
                  pl.BlockSpec((MB, D), lambda m: (m + GM, 0)),
                  pl.BlockSpec((MB, D), lambda m: (m, 0)),
                  pl.BlockSpec((D, C), lambda m: (0, 0))],
        out_specs=pl.BlockSpec((MB, C), lambda m: (m, 0)),
        out_shape=jax.ShapeDtypeStruct((N, C), jnp.float32),
    )(agg_flat, agg_flat, hr_prev, cls_w)


@functools.cache
def _mesh():
    return plsc.VectorSubcoreMesh(core_axis_name="c", subcore_axis_name="s",
                                  num_cores=NC, num_subcores=NS)


def _norm_edges(src2, dst2, et2):
    @functools.partial(
        pl.kernel,
        out_type=[jax.ShapeDtypeStruct((EROWS, CHW), jnp.float32),
                  jax.ShapeDtypeStruct((EROWS, CHW), jnp.int32)],
        mesh=_mesh(),
        compiler_params=pltpu.CompilerParams(use_tc_tiling_on_sc=False, needs_layout_passes=False),
        scratch_types=[
            pltpu.VMEM_SHARED((NR,), jnp.float32),
            pltpu.VMEM((NR,), jnp.float32),
            pltpu.VMEM((PB, CHW), jnp.int32),
            pltpu.VMEM((PB, CHW), jnp.int32),
            pltpu.VMEM((PB, CHW), jnp.int32),
            pltpu.VMEM((PB, CHW), jnp.int32),
            pltpu.VMEM((PB, CHW), jnp.float32),
            pltpu.VMEM((PB, CHW), jnp.int32),
            pltpu.VMEM((CHW,), jnp.float32),
            pltpu.VMEM((PB * CHW,), jnp.float32),
        ],
    )
    def k(src_h, dst_h, et_h, norm_h, gidx_h,
          cnt_sh, cnt_v, dst_v, et_v, src_v, keyv_v, norm_v, gidx_v, ones_v,
          zf_v):
        sid = lax.axis_index("s")
        cid = lax.axis_index("c")

        for j in range(CHW // LANES):
            ones_v[pl.ds(j * LANES, LANES)] = jnp.ones((LANES,), jnp.float32)

        zwords = PB * CHW

        def zrow(i, _):
            zf_v[pl.ds(i * LANES, LANES)] = jnp.zeros((LANES,), jnp.float32)
            return 0
        lax.fori_loop(0, zwords // LANES, zrow, 0)
        for t in range(3):
            chunk_id = sid + NS * t

            @pl.when(chunk_id < NR // zwords)
            def _():
                pltpu.sync_copy(zf_v,
                                cnt_sh.at[pl.ds(chunk_id * zwords, zwords)])
        plsc.subcore_barrier()

        rows_all = EROWS // NS

        def count_pass(p, _):
            base = sid * rows_all + p * PB
            pltpu.sync_copy(dst_h.at[pl.ds(base, PB)], dst_v)
            pltpu.sync_copy(et_h.at[pl.ds(base, PB)], et_v)

            def kvrow(i, _2):
                for j in range(CHW // LANES):
                    dd = dst_v[i, pl.ds(j * LANES, LANES)]
                    tt = et_v[i, pl.ds(j * LANES, LANES)]
                    keyv_v[i, pl.ds(j * LANES, LANES)] = dd * R + tt
                return 0
            lax.fori_loop(0, PB, kvrow, 0)

            def scat(i, _2):
                pltpu.sync_copy(ones_v, cnt_sh.at[keyv_v.at[i]], add=True)
                return 0
            lax.fori_loop(0, PB, scat, 0)
            return 0
        lax.fori_loop(0, rows_all // PB, count_pass, 0)
        plsc.subcore_barrier()

        pltpu.sync_copy(cnt_sh, cnt_v)
        wbase = (cid * NS + sid) * ROWS_W

        for p in range(ROWS_W // PB):
            base = wbase + p * PB
            pltpu.sync_copy(dst_h.at[pl.ds(base, PB)], dst_v)
            pltpu.sync_copy(et_h.at[pl.ds(base, PB)], et_v)
            pltpu.sync_copy(src_h.at[pl.ds(base, PB)], src_v)

            def nrow(i, _2):
                for j in range(CHW // LANES):
                    dd = dst_v[i, pl.ds(j * LANES, LANES)]
                    tt = et_v[i, pl.ds(j * LANES, LANES)]
                    ss = src_v[i, pl.ds(j * LANES, LANES)]
                    cnt = plsc.load_gather(cnt_v, [dd * R + tt])
                    norm_v[i, pl.ds(j * LANES, LANES)] = (
                        1.0 / jnp.maximum(cnt, 1.0))
                    gidx_v[i, pl.ds(j * LANES, LANES)] = tt * N + ss
                return 0
            lax.fori_loop(0, PB, nrow, 0)
            pltpu.sync_copy(norm_v, norm_h.at[pl.ds(base, PB)])
            pltpu.sync_copy(gidx_v, gidx_h.at[pl.ds(base, PB)])

    return k(src2, dst2, et2)


_ZROWS = 25


def _edge_scatter(xwf, gidx2, norm2, dst2):
    @functools.partial(
        pl.kernel,
        out_type=jax.ShapeDtypeStruct((NC * N, D), jnp.float32),
        mesh=_mesh(),
        compiler_params=pltpu.CompilerParams(use_tc_tiling_on_sc=False, needs_layout_passes=False),
        scratch_types=[
            pltpu.VMEM_SHARED((N, D), jnp.float32),
            pltpu.VMEM((ROWS_W, CHW), jnp.int32),
            pltpu.VMEM((ROWS_W, CHW), jnp.float32),
            pltpu.VMEM((ROWS_W, CHW), jnp.int32),
            pltpu.VMEM((CHW, D), jnp.float32),
            pltpu.VMEM((_ZROWS, D), jnp.float32),
            pltpu.SemaphoreType.DMA,
        ],
    )
    def k(xw_h, gi_h, no_h, ds_h, out_h,
          agg_sh, gi_v, no_v, ds_v, rows_v, z_v, sem):
        sid = lax.axis_index("s")
        cid = lax.axis_index("c")
        nrows = N // NS

        def zrow(i, _):
            for j in range(D // LANES):
                z_v[i, pl.ds(j * LANES, LANES)] = jnp.zeros((LANES,),
                                                            jnp.float32)
            return 0
        lax.fori_loop(0, _ZROWS, zrow, 0)
        for t in range(nrows // _ZROWS):
            pltpu.sync_copy(
                z_v, agg_sh.at[pl.ds(sid * nrows + t * _ZROWS, _ZROWS)])
        plsc.subcore_barrier()

        w = cid * NS + sid
        pltpu.sync_copy(gi_h.at[pl.ds(w * ROWS_W, ROWS_W)], gi_v)
        pltpu.sync_copy(no_h.at[pl.ds(w * ROWS_W, ROWS_W)], no_v)
        pltpu.sync_copy(ds_h.at[pl.ds(w * ROWS_W, ROWS_W)], ds_v)

        def chunk(i, _):
            pltpu.async_copy(xw_h.at[gi_v.at[i]], rows_v, sem).wait()
            iv = jnp.full((LANES,), i, jnp.int32)

            def srow(e, _2):
                nv = plsc.load_gather(no_v, [iv, jnp.full((LANES,), e,
                                                          jnp.int32)])
                for j in range(D // LANES):
                    rows_v[e, pl.ds(j * LANES, LANES)] = (
                        rows_v[e, pl.ds(j * LANES, LANES)] * nv)
                return 0
            lax.fori_loop(0, CHW, srow, 0)
            pltpu.sync_copy(rows_v, agg_sh.at[ds_v.at[i]], add=True)
            return 0
        lax.fori_loop(0, ROWS_W, chunk, 0)
        plsc.subcore_barrier()
        pltpu.sync_copy(agg_sh.at[pl.ds(sid * nrows, nrows)],
                        out_h.at[pl.ds(cid * N + sid * nrows, nrows)])

    return k(xwf, gidx2, norm2, dst2)


def kernel(x, edge_index, edge_type, emb_w, basis1, comp1, root1, bias1,
           basis2, comp2, root2, bias2, cls_w):
    src2 = edge_index[0].reshape(EROWS, CHW)
    dst2 = edge_index[1].reshape(EROWS, CHW)
    et2 = edge_type.reshape(EROWS, CHW)

    wf1 = _wmix(comp1, basis1.reshape(R, D * D),
                root1.reshape(1, D * D)).reshape(R + 1, D, D)
    wf2 = _wmix(comp2, basis2.reshape(R, D * D),
                root2.reshape(1, D * D)).reshape(R + 1, D, D)

    norm2, gidx2 = _norm_edges(src2, dst2, et2)

    h0 = _embed_mm(x, emb_w)
    xw1, hr1 = _layer_mm(h0, wf1, bias1.reshape(1, D))
    agg1 = _edge_scatter(xw1.reshape(R * N, D), gidx2, norm2, dst2)
    xw2, hr2 = _layer_mm2(agg1, hr1, wf2, bias2.reshape(1, D))
    agg2 = _edge_scatter(xw2.reshape(R * N, D), gidx2, norm2, dst2)
    return _decode(agg2, hr2, cls_w)
